# baseline (speedup 1.0000x reference)
import functools

import jax
import jax.numpy as jnp
from jax import lax
from jax.experimental import pallas as pl
from jax.experimental.pallas import tpu as pltpu
from jax.experimental.pallas import tpu_sc as plsc

N = 10000
E = 320000
D = 128
H = 128

NPAD = 10240
CH = 128
NC = 2
NS = 16
NW = NC * NS
K = 80
EPAD = NW * K * CH
NBUF = 2
G = 16
NG = K // G
ROWS_PER_TILE = NPAD // NS

f32 = jnp.float32
i32 = jnp.int32

_MESH = plsc.VectorSubcoreMesh(core_axis_name="c", subcore_axis_name="s")


def _common_prologue(src_hbm, zeros2d_hbm, src_v, acc_sh, cid, sid):
    wid = cid * NS + sid
    r0 = sid * ROWS_PER_TILE
    pltpu.sync_copy(zeros2d_hbm.at[pl.ds(r0, ROWS_PER_TILE)],
                    acc_sh.at[pl.ds(r0, ROWS_PER_TILE)])
    pltpu.sync_copy(src_hbm.at[wid], src_v)


def _gather_scatter_loop(feat_hbm, dst_hbm, wid, src_v, dst_g, rows_v,
                         acc_sh, gsem, chunk_extra):
    for b in range(NBUF):
        pltpu.async_copy(feat_hbm.at[src_v.at[b]], rows_v.at[b], gsem.at[b])

    def group(g, carry):
        pltpu.sync_copy(dst_hbm.at[wid].at[pl.ds(g * G, G)], dst_g)

        def pair(p, carry2):
            for b in range(NBUF):
                u = p * NBUF + b
                j = g * G + u
                pltpu.make_async_copy(feat_hbm.at[src_v.at[j]],
                                      rows_v.at[b], gsem.at[b]).wait()
                if chunk_extra is not None:
                    chunk_extra(u)
                pltpu.sync_copy(rows_v.at[b], acc_sh.at[dst_g.at[u]],
                                add=True)
                pltpu.async_copy(feat_hbm.at[src_v.at[j + NBUF]],
                                 rows_v.at[b], gsem.at[b])
            return carry2

        lax.fori_loop(0, G // NBUF, pair, 0)
        return carry

    lax.fori_loop(0, NG, group, 0)
    for b in range(NBUF):
        pltpu.make_async_copy(feat_hbm.at[src_v.at[K + b]], rows_v.at[b],
                              gsem.at[b]).wait()


@functools.partial(
    pl.kernel,
    out_type=[jax.ShapeDtypeStruct((NC, NPAD, D), f32),
              jax.ShapeDtypeStruct((NC, NPAD), f32)],
    mesh=_MESH,
    scratch_types=[
        pltpu.VMEM((K + NBUF, CH), i32),
        pltpu.VMEM((G, CH), i32),
        pltpu.VMEM((NBUF, CH, D), f32),
        pltpu.VMEM((CH,), f32),
        pltpu.VMEM_SHARED((NPAD, D), f32),
        pltpu.VMEM_SHARED((NPAD,), f32),
        pltpu.SemaphoreType.DMA((NBUF,)),
    ],
    name="sc_segsum_deg",
)
def _sc_segsum_deg(feat_hbm, src_hbm, dst_hbm, zeros2d_hbm, zeros1d_hbm,
                   acc_out, deg_out,
                   src_v, dst_g, rows_v, ones_v, acc_sh, deg_sh, gsem):
    cid = lax.axis_index("c")
    sid = lax.axis_index("s")
    wid = cid * NS + sid
    r0 = sid * ROWS_PER_TILE
    _common_prologue(src_hbm, zeros2d_hbm, src_v, acc_sh, cid, sid)
    pltpu.sync_copy(zeros1d_hbm.at[pl.ds(r0, ROWS_PER_TILE)],
                    deg_sh.at[pl.ds(r0, ROWS_PER_TILE)])
    for i in range(CH // 16):
        ones_v[pl.ds(i * 16, 16)] = jnp.ones((16,), f32)
    plsc.subcore_barrier()

    def count_deg(u):
        pltpu.sync_copy(ones_v, deg_sh.at[dst_g.at[u]], add=True)

    _gather_scatter_loop(feat_hbm, dst_hbm, wid, src_v, dst_g, rows_v,
                         acc_sh, gsem, count_deg)
    plsc.subcore_barrier()
    pltpu.sync_copy(acc_sh.at[pl.ds(r0, ROWS_PER_TILE)],
                    acc_out.at[cid].at[pl.ds(r0, ROWS_PER_TILE)])
    pltpu.sync_copy(deg_sh.at[pl.ds(r0, ROWS_PER_TILE)],
                    deg_out.at[cid].at[pl.ds(r0, ROWS_PER_TILE)])


@functools.partial(
    pl.kernel,
    out_type=[jax.ShapeDtypeStruct((NC, NPAD, D), f32)],
    mesh=_MESH,
    scratch_types=[
        pltpu.VMEM((K + NBUF, CH), i32),
        pltpu.VMEM((G, CH), i32),
        pltpu.VMEM((NBUF, CH, D), f32),
        pltpu.VMEM_SHARED((NPAD, D), f32),
        pltpu.SemaphoreType.DMA((NBUF,)),
    ],
    name="sc_segsum",
)
def _sc_segsum(feat_hbm, src_hbm, dst_hbm, zeros2d_hbm,
               acc_out,
               src_v, dst_g, rows_v, acc_sh, gsem):
    cid = lax.axis_index("c")
    sid = lax.axis_index("s")
    wid = cid * NS + sid
    r0 = sid * ROWS_PER_TILE
    _common_prologue(src_hbm, zeros2d_hbm, src_v, acc_sh, cid, sid)
    plsc.subcore_barrier()
    _gather_scatter_loop(feat_hbm, dst_hbm, wid, src_v, dst_g, rows_v,
                         acc_sh, gsem, None)
    plsc.subcore_barrier()
    pltpu.sync_copy(acc_sh.at[pl.ds(r0, ROWS_PER_TILE)],
                    acc_out.at[cid].at[pl.ds(r0, ROWS_PER_TILE)])


RB = 1024


def _tc_body(relu, p0_ref, p1_ref, degt_ref, x_ref, wl_ref, wr_ref, b_ref,
             o_ref):
    deg = degt_ref[:, 0:1] + degt_ref[:, 1:2]
    inv = 1.0 / jnp.maximum(deg, 1.0)
    agg = (p0_ref[0] + p1_ref[0]) * inv
    y = (jnp.dot(agg, wl_ref[...], preferred_element_type=f32)
         + jnp.dot(x_ref[...], wr_ref[...], preferred_element_type=f32)
         + b_ref[...])
    if relu:
        y = jnp.maximum(y, 0.0)
    o_ref[...] = y


def _tc_layer(parts, degt, x_pad, W_l, W_r, b, relu):
    return pl.pallas_call(
        functools.partial(_tc_body, relu),
        grid=(NPAD // RB,),
        in_specs=[
            pl.BlockSpec((1, RB, D), lambda i: (0, i, 0)),
            pl.BlockSpec((1, RB, D), lambda i: (1, i, 0)),
            pl.BlockSpec((RB, 2), lambda i: (i, 0)),
            pl.BlockSpec((RB, D), lambda i: (i, 0)),
            pl.BlockSpec((D, H), lambda i: (0, 0)),
            pl.BlockSpec((D, H), lambda i: (0, 0)),
            pl.BlockSpec((1, H), lambda i: (0, 0)),
        ],
        out_specs=pl.BlockSpec((RB, H), lambda i: (i, 0)),
        out_shape=jax.ShapeDtypeStruct((NPAD, H), f32),
    )(parts, parts, degt, x_pad, W_l, W_r, b.reshape(1, H))


def kernel(x, edge_index, W_l0, W_r0, b0, W_l1, W_r1, b1):
    src = edge_index[0]
    dst = edge_index[1]
    pad = EPAD - E
    src_p = jnp.concatenate([src, jnp.arange(pad, dtype=i32) % N])
    dst_p = jnp.concatenate([dst, N + jnp.arange(pad, dtype=i32) % (NPAD - N)])
    src3 = jnp.concatenate(
        [src_p.reshape(NW, K, CH), jnp.zeros((NW, NBUF, CH), i32)], axis=1)
    dst3 = dst_p.reshape(NW, K, CH)
    x_pad = jnp.concatenate([x, jnp.zeros((NPAD - N, D), f32)])
    z2 = jnp.zeros((NPAD, D), f32)
    z1 = jnp.zeros((NPAD,), f32)

    acc0, deg = _sc_segsum_deg(x_pad, src3, dst3, z2, z1)
    degt = deg.T
    h_pad = _tc_layer(acc0, degt, x_pad, W_l0, W_r0, b0, relu=True)
    (acc1,) = _sc_segsum(h_pad, src3, dst3, z2)
    out_pad = _tc_layer(acc1, degt, h_pad, W_l1, W_r1, b1, relu=False)
    return out_pad[:N]


# speedup vs baseline: 2.0508x; 2.0508x over previous
import functools

import jax
import jax.numpy as jnp
from jax import lax
from jax.experimental import pallas as pl
from jax.experimental.pallas import tpu as pltpu
from jax.experimental.pallas import tpu_sc as plsc

N = 10000
E = 320000
D = 128
H = 128

NPAD = 10240
CH = 128
NC = 2
NS = 16
NW = NC * NS
K = 80
EPAD = NW * K * CH
NBUF = 1
G = 16
NG = K // G
ROWS_PER_TILE = NPAD // NS

f32 = jnp.float32
i32 = jnp.int32

_MESH = plsc.VectorSubcoreMesh(core_axis_name="c", subcore_axis_name="s")


def _common_prologue(src_hbm, zeros2d_hbm, src_v, acc_sh, cid, sid):
    wid = cid * NS + sid
    r0 = sid * ROWS_PER_TILE
    pltpu.sync_copy(zeros2d_hbm.at[pl.ds(r0, ROWS_PER_TILE)],
                    acc_sh.at[pl.ds(r0, ROWS_PER_TILE)])
    pltpu.sync_copy(src_hbm.at[wid], src_v)


def _gather_scatter_loop(feat_hbm, dst_hbm, wid, src_v, dst_g, rows_v,
                         acc_sh, gsem, chunk_extra):
    def group(g, carry):
        pltpu.sync_copy(dst_hbm.at[wid].at[pl.ds(g * G, G)], dst_g)

        def body(u, carry2):
            j = g * G + u
            pltpu.async_copy(feat_hbm.at[src_v.at[j]], rows_v.at[0],
                             gsem.at[0]).wait()
            if chunk_extra is not None:
                chunk_extra(u)
            pltpu.sync_copy(rows_v.at[0], acc_sh.at[dst_g.at[u]], add=True)
            return carry2

        lax.fori_loop(0, G, body, 0)
        return carry

    lax.fori_loop(0, NG, group, 0)


@functools.partial(
    pl.kernel,
    out_type=[jax.ShapeDtypeStruct((NC, NPAD, D), f32),
              jax.ShapeDtypeStruct((NC, NPAD), f32)],
    mesh=_MESH,
    scratch_types=[
        pltpu.VMEM((K + NBUF, CH), i32),
        pltpu.VMEM((G, CH), i32),
        pltpu.VMEM((NBUF, CH, D), f32),
        pltpu.VMEM((CH,), f32),
        pltpu.VMEM_SHARED((NPAD, D), f32),
        pltpu.VMEM_SHARED((NPAD,), f32),
        pltpu.SemaphoreType.DMA((NBUF,)),
    ],
    name="sc_segsum_deg",
)
def _sc_segsum_deg(feat_hbm, src_hbm, dst_hbm, zeros2d_hbm, zeros1d_hbm,
                   acc_out, deg_out,
                   src_v, dst_g, rows_v, ones_v, acc_sh, deg_sh, gsem):
    cid = lax.axis_index("c")
    sid = lax.axis_index("s")
    wid = cid * NS + sid
    r0 = sid * ROWS_PER_TILE
    _common_prologue(src_hbm, zeros2d_hbm, src_v, acc_sh, cid, sid)
    pltpu.sync_copy(zeros1d_hbm.at[pl.ds(r0, ROWS_PER_TILE)],
                    deg_sh.at[pl.ds(r0, ROWS_PER_TILE)])
    for i in range(CH // 16):
        ones_v[pl.ds(i * 16, 16)] = jnp.ones((16,), f32)
    plsc.subcore_barrier()

    def count_deg(u):
        pltpu.sync_copy(ones_v, deg_sh.at[dst_g.at[u]], add=True)

    _gather_scatter_loop(feat_hbm, dst_hbm, wid, src_v, dst_g, rows_v,
                         acc_sh, gsem, count_deg)
    plsc.subcore_barrier()
    pltpu.sync_copy(acc_sh.at[pl.ds(r0, ROWS_PER_TILE)],
                    acc_out.at[cid].at[pl.ds(r0, ROWS_PER_TILE)])
    pltpu.sync_copy(deg_sh.at[pl.ds(r0, ROWS_PER_TILE)],
                    deg_out.at[cid].at[pl.ds(r0, ROWS_PER_TILE)])


@functools.partial(
    pl.kernel,
    out_type=[jax.ShapeDtypeStruct((NC, NPAD, D), f32)],
    mesh=_MESH,
    scratch_types=[
        pltpu.VMEM((K + NBUF, CH), i32),
        pltpu.VMEM((G, CH), i32),
        pltpu.VMEM((NBUF, CH, D), f32),
        pltpu.VMEM_SHARED((NPAD, D), f32),
        pltpu.SemaphoreType.DMA((NBUF,)),
    ],
    name="sc_segsum",
)
def _sc_segsum(feat_hbm, src_hbm, dst_hbm, zeros2d_hbm,
               acc_out,
               src_v, dst_g, rows_v, acc_sh, gsem):
    cid = lax.axis_index("c")
    sid = lax.axis_index("s")
    wid = cid * NS + sid
    r0 = sid * ROWS_PER_TILE
    _common_prologue(src_hbm, zeros2d_hbm, src_v, acc_sh, cid, sid)
    plsc.subcore_barrier()
    _gather_scatter_loop(feat_hbm, dst_hbm, wid, src_v, dst_g, rows_v,
                         acc_sh, gsem, None)
    plsc.subcore_barrier()
    pltpu.sync_copy(acc_sh.at[pl.ds(r0, ROWS_PER_TILE)],
                    acc_out.at[cid].at[pl.ds(r0, ROWS_PER_TILE)])


RB = 1024


def _tc_body(relu, p0_ref, p1_ref, degt_ref, x_ref, wl_ref, wr_ref, b_ref,
             o_ref):
    deg = degt_ref[:, 0:1] + degt_ref[:, 1:2]
    inv = 1.0 / jnp.maximum(deg, 1.0)
    agg = (p0_ref[0] + p1_ref[0]) * inv
    y = (jnp.dot(agg, wl_ref[...], preferred_element_type=f32)
         + jnp.dot(x_ref[...], wr_ref[...], preferred_element_type=f32)
         + b_ref[...])
    if relu:
        y = jnp.maximum(y, 0.0)
    o_ref[...] = y


def _tc_layer(parts, degt, x_pad, W_l, W_r, b, relu):
    return pl.pallas_call(
        functools.partial(_tc_body, relu),
        grid=(NPAD // RB,),
        in_specs=[
            pl.BlockSpec((1, RB, D), lambda i: (0, i, 0)),
            pl.BlockSpec((1, RB, D), lambda i: (1, i, 0)),
            pl.BlockSpec((RB, 2), lambda i: (i, 0)),
            pl.BlockSpec((RB, D), lambda i: (i, 0)),
            pl.BlockSpec((D, H), lambda i: (0, 0)),
            pl.BlockSpec((D, H), lambda i: (0, 0)),
            pl.BlockSpec((1, H), lambda i: (0, 0)),
        ],
        out_specs=pl.BlockSpec((RB, H), lambda i: (i, 0)),
        out_shape=jax.ShapeDtypeStruct((NPAD, H), f32),
    )(parts, parts, degt, x_pad, W_l, W_r, b.reshape(1, H))


def kernel(x, edge_index, W_l0, W_r0, b0, W_l1, W_r1, b1):
    src = edge_index[0]
    dst = edge_index[1]
    pad = EPAD - E
    src_p = jnp.concatenate([src, jnp.arange(pad, dtype=i32) % N])
    dst_p = jnp.concatenate([dst, N + jnp.arange(pad, dtype=i32) % (NPAD - N)])
    src3 = jnp.concatenate(
        [src_p.reshape(NW, K, CH), jnp.zeros((NW, NBUF, CH), i32)], axis=1)
    dst3 = dst_p.reshape(NW, K, CH)
    x_pad = jnp.concatenate([x, jnp.zeros((NPAD - N, D), f32)])
    z2 = jnp.zeros((NPAD, D), f32)
    z1 = jnp.zeros((NPAD,), f32)

    acc0, deg = _sc_segsum_deg(x_pad, src3, dst3, z2, z1)
    degt = deg.T
    h_pad = _tc_layer(acc0, degt, x_pad, W_l0, W_r0, b0, relu=True)
    (acc1,) = _sc_segsum(h_pad, src3, dst3, z2)
    out_pad = _tc_layer(acc1, degt, h_pad, W_l1, W_r1, b1, relu=False)
    return out_pad[:N]


# speedup vs baseline: 2.0891x; 1.0187x over previous
import functools

import jax
import jax.numpy as jnp
from jax import lax
from jax.experimental import pallas as pl
from jax.experimental.pallas import tpu as pltpu
from jax.experimental.pallas import tpu_sc as plsc

N = 10000
E = 320000
D = 128
H = 128

NPAD = 10240
CH = 128
NC = 2
NS = 16
NW = NC * NS
K = 80
EPAD = NW * K * CH
NBUF = 1
G = 16
NG = K // G
ROWS_PER_TILE = NPAD // NS

f32 = jnp.float32
i32 = jnp.int32

_MESH = plsc.VectorSubcoreMesh(core_axis_name="c", subcore_axis_name="s")


def _common_prologue(src_hbm, zeros2d_hbm, src_v, acc_sh, cid, sid):
    wid = cid * NS + sid
    r0 = sid * ROWS_PER_TILE
    pltpu.sync_copy(zeros2d_hbm.at[pl.ds(r0, ROWS_PER_TILE)],
                    acc_sh.at[pl.ds(r0, ROWS_PER_TILE)])
    pltpu.sync_copy(src_hbm.at[wid], src_v)


def _gather_scatter_loop(feat_hbm, dst_hbm, wid, src_v, dst_g, rows_v,
                         acc_sh, gsem, chunk_extra):
    def group(g, carry):
        pltpu.sync_copy(dst_hbm.at[wid].at[pl.ds(g * G, G)], dst_g)

        def body(u, carry2):
            j = g * G + u
            pltpu.async_copy(feat_hbm.at[src_v.at[j]], rows_v.at[0],
                             gsem.at[0]).wait()
            if chunk_extra is not None:
                chunk_extra(u)
            pltpu.sync_copy(rows_v.at[0], acc_sh.at[dst_g.at[u]], add=True)
            return carry2

        lax.fori_loop(0, G, body, 0)
        return carry

    lax.fori_loop(0, NG, group, 0)


@functools.partial(
    pl.kernel,
    out_type=[jax.ShapeDtypeStruct((NC, NPAD, D), f32),
              jax.ShapeDtypeStruct((NC, NPAD), f32)],
    mesh=_MESH,
    scratch_types=[
        pltpu.VMEM((K + NBUF, CH), i32),
        pltpu.VMEM((G, CH), i32),
        pltpu.VMEM((NBUF, CH, D), f32),
        pltpu.VMEM((CH,), f32),
        pltpu.VMEM_SHARED((NPAD, D), f32),
        pltpu.VMEM_SHARED((NPAD,), f32),
        pltpu.SemaphoreType.DMA((NBUF,)),
    ],
    name="sc_segsum_deg",
)
def _sc_segsum_deg(feat_hbm, src_hbm, dst_hbm, zeros2d_hbm, zeros1d_hbm,
                   acc_out, deg_out,
                   src_v, dst_g, rows_v, ones_v, acc_sh, deg_sh, gsem):
    cid = lax.axis_index("c")
    sid = lax.axis_index("s")
    wid = cid * NS + sid
    r0 = sid * ROWS_PER_TILE
    _common_prologue(src_hbm, zeros2d_hbm, src_v, acc_sh, cid, sid)
    pltpu.sync_copy(zeros1d_hbm.at[pl.ds(r0, ROWS_PER_TILE)],
                    deg_sh.at[pl.ds(r0, ROWS_PER_TILE)])
    for i in range(CH // 16):
        ones_v[pl.ds(i * 16, 16)] = jnp.ones((16,), f32)
    plsc.subcore_barrier()

    def count_deg(u):
        pltpu.sync_copy(ones_v, deg_sh.at[dst_g.at[u]], add=True)

    _gather_scatter_loop(feat_hbm, dst_hbm, wid, src_v, dst_g, rows_v,
                         acc_sh, gsem, count_deg)
    plsc.subcore_barrier()
    pltpu.sync_copy(acc_sh.at[pl.ds(r0, ROWS_PER_TILE)],
                    acc_out.at[cid].at[pl.ds(r0, ROWS_PER_TILE)])
    pltpu.sync_copy(deg_sh.at[pl.ds(r0, ROWS_PER_TILE)],
                    deg_out.at[cid].at[pl.ds(r0, ROWS_PER_TILE)])


@functools.partial(
    pl.kernel,
    out_type=[jax.ShapeDtypeStruct((NC, NPAD, D), f32)],
    mesh=_MESH,
    scratch_types=[
        pltpu.VMEM((K + NBUF, CH), i32),
        pltpu.VMEM((G, CH), i32),
        pltpu.VMEM((NBUF, CH, D), f32),
        pltpu.VMEM_SHARED((NPAD, D), f32),
        pltpu.SemaphoreType.DMA((NBUF,)),
    ],
    name="sc_segsum",
)
def _sc_segsum(feat_hbm, src_hbm, dst_hbm, zeros2d_hbm,
               acc_out,
               src_v, dst_g, rows_v, acc_sh, gsem):
    cid = lax.axis_index("c")
    sid = lax.axis_index("s")
    wid = cid * NS + sid
    r0 = sid * ROWS_PER_TILE
    _common_prologue(src_hbm, zeros2d_hbm, src_v, acc_sh, cid, sid)
    plsc.subcore_barrier()
    _gather_scatter_loop(feat_hbm, dst_hbm, wid, src_v, dst_g, rows_v,
                         acc_sh, gsem, None)
    plsc.subcore_barrier()
    pltpu.sync_copy(acc_sh.at[pl.ds(r0, ROWS_PER_TILE)],
                    acc_out.at[cid].at[pl.ds(r0, ROWS_PER_TILE)])


RB = 1000


def _tc_body(relu, p0_ref, p1_ref, degt_ref, x_ref, wl_ref, wr_ref, b_ref,
             o_ref):
    deg = degt_ref[:, 0:1] + degt_ref[:, 1:2]
    inv = 1.0 / jnp.maximum(deg, 1.0)
    agg = (p0_ref[0] + p1_ref[0]) * inv
    y = (jnp.dot(agg, wl_ref[...], preferred_element_type=f32)
         + jnp.dot(x_ref[...], wr_ref[...], preferred_element_type=f32)
         + b_ref[...])
    if relu:
        y = jnp.maximum(y, 0.0)
    o_ref[...] = y


def _tc_layer(parts, degt, x_in, W_l, W_r, b, relu):
    return pl.pallas_call(
        functools.partial(_tc_body, relu),
        grid=(N // RB,),
        in_specs=[
            pl.BlockSpec((1, RB, D), lambda i: (0, i, 0)),
            pl.BlockSpec((1, RB, D), lambda i: (1, i, 0)),
            pl.BlockSpec((RB, 2), lambda i: (i, 0)),
            pl.BlockSpec((RB, D), lambda i: (i, 0)),
            pl.BlockSpec((D, H), lambda i: (0, 0)),
            pl.BlockSpec((D, H), lambda i: (0, 0)),
            pl.BlockSpec((1, H), lambda i: (0, 0)),
        ],
        out_specs=pl.BlockSpec((RB, H), lambda i: (i, 0)),
        out_shape=jax.ShapeDtypeStruct((N, H), f32),
    )(parts, parts, degt, x_in, W_l, W_r, b.reshape(1, H))


def kernel(x, edge_index, W_l0, W_r0, b0, W_l1, W_r1, b1):
    src = edge_index[0]
    dst = edge_index[1]
    pad = EPAD - E
    src_p = jnp.concatenate([src, jnp.arange(pad, dtype=i32) % N])
    dst_p = jnp.concatenate([dst, N + jnp.arange(pad, dtype=i32) % (NPAD - N)])
    src3 = jnp.concatenate(
        [src_p.reshape(NW, K, CH), jnp.zeros((NW, NBUF, CH), i32)], axis=1)
    dst3 = dst_p.reshape(NW, K, CH)
    z2 = jnp.zeros((NPAD, D), f32)
    z1 = jnp.zeros((NPAD,), f32)

    acc0, deg = _sc_segsum_deg(x, src3, dst3, z2, z1)
    degt = deg.T
    h = _tc_layer(acc0, degt, x, W_l0, W_r0, b0, relu=True)
    (acc1,) = _sc_segsum(h, src3, dst3, z2)
    return _tc_layer(acc1, degt, h, W_l1, W_r1, b1, relu=False)


# speedup vs baseline: 2.0914x; 1.0011x over previous
import functools

import jax
import jax.numpy as jnp
from jax import lax
from jax.experimental import pallas as pl
from jax.experimental.pallas import tpu as pltpu
from jax.experimental.pallas import tpu_sc as plsc

N = 10000
E = 320000
D = 128
H = 128

NPAD = 10240
CH = 128
NC = 2
NS = 16
NW = NC * NS
K = 80
EPAD = NW * K * CH
MC = 1
CB = MC * CH
KC = K // MC
G = 16
NG = KC // G
ROWS_PER_TILE = NPAD // NS

f32 = jnp.float32
i32 = jnp.int32

_MESH = plsc.VectorSubcoreMesh(core_axis_name="c", subcore_axis_name="s")


def _common_prologue(src_hbm, zeros2d_hbm, src_v, acc_sh, cid, sid):
    wid = cid * NS + sid
    r0 = sid * ROWS_PER_TILE
    pltpu.sync_copy(zeros2d_hbm.at[pl.ds(r0, ROWS_PER_TILE)],
                    acc_sh.at[pl.ds(r0, ROWS_PER_TILE)])
    pltpu.sync_copy(src_hbm.at[wid], src_v)


def _gather_scatter_loop(feat_hbm, dst_hbm, wid, src_v, dst_g, rows_v,
                         acc_sh, gsem, chunk_extra):
    def group(g, carry):
        pltpu.sync_copy(dst_hbm.at[wid].at[pl.ds(g * G, G)], dst_g)

        def body(u, carry2):
            j = g * G + u
            pltpu.async_copy(feat_hbm.at[src_v.at[j]], rows_v, gsem).wait()
            if chunk_extra is not None:
                chunk_extra(u)
            pltpu.sync_copy(rows_v, acc_sh.at[dst_g.at[u]], add=True)
            return carry2

        lax.fori_loop(0, G, body, 0)
        return carry

    lax.fori_loop(0, NG, group, 0)


@functools.partial(
    pl.kernel,
    out_type=[jax.ShapeDtypeStruct((NC, NPAD, D), f32),
              jax.ShapeDtypeStruct((NC, NPAD), f32)],
    mesh=_MESH,
    scratch_types=[
        pltpu.VMEM((KC, CB), i32),
        pltpu.VMEM((G, CB), i32),
        pltpu.VMEM((CB, D), f32),
        pltpu.VMEM((CB,), f32),
        pltpu.VMEM_SHARED((NPAD, D), f32),
        pltpu.VMEM_SHARED((NPAD,), f32),
        pltpu.SemaphoreType.DMA,
    ],
    name="sc_segsum_deg",
)
def _sc_segsum_deg(feat_hbm, src_hbm, dst_hbm, zeros2d_hbm, zeros1d_hbm,
                   acc_out, deg_out,
                   src_v, dst_g, rows_v, ones_v, acc_sh, deg_sh, gsem):
    cid = lax.axis_index("c")
    sid = lax.axis_index("s")
    wid = cid * NS + sid
    r0 = sid * ROWS_PER_TILE
    _common_prologue(src_hbm, zeros2d_hbm, src_v, acc_sh, cid, sid)
    pltpu.sync_copy(zeros1d_hbm.at[pl.ds(r0, ROWS_PER_TILE)],
                    deg_sh.at[pl.ds(r0, ROWS_PER_TILE)])
    for i in range(CB // 16):
        ones_v[pl.ds(i * 16, 16)] = jnp.ones((16,), f32)
    plsc.subcore_barrier()

    def count_deg(u):
        pltpu.sync_copy(ones_v, deg_sh.at[dst_g.at[u]], add=True)

    _gather_scatter_loop(feat_hbm, dst_hbm, wid, src_v, dst_g, rows_v,
                         acc_sh, gsem, count_deg)
    plsc.subcore_barrier()
    pltpu.sync_copy(acc_sh.at[pl.ds(r0, ROWS_PER_TILE)],
                    acc_out.at[cid].at[pl.ds(r0, ROWS_PER_TILE)])
    pltpu.sync_copy(deg_sh.at[pl.ds(r0, ROWS_PER_TILE)],
                    deg_out.at[cid].at[pl.ds(r0, ROWS_PER_TILE)])


@functools.partial(
    pl.kernel,
    out_type=[jax.ShapeDtypeStruct((NC, NPAD, D), f32)],
    mesh=_MESH,
    scratch_types=[
        pltpu.VMEM((KC, CB), i32),
        pltpu.VMEM((G, CB), i32),
        pltpu.VMEM((CB, D), f32),
        pltpu.VMEM_SHARED((NPAD, D), f32),
        pltpu.SemaphoreType.DMA,
    ],
    name="sc_segsum",
)
def _sc_segsum(feat_hbm, src_hbm, dst_hbm, zeros2d_hbm,
               acc_out,
               src_v, dst_g, rows_v, acc_sh, gsem):
    cid = lax.axis_index("c")
    sid = lax.axis_index("s")
    wid = cid * NS + sid
    r0 = sid * ROWS_PER_TILE
    _common_prologue(src_hbm, zeros2d_hbm, src_v, acc_sh, cid, sid)
    plsc.subcore_barrier()
    _gather_scatter_loop(feat_hbm, dst_hbm, wid, src_v, dst_g, rows_v,
                         acc_sh, gsem, None)
    plsc.subcore_barrier()
    pltpu.sync_copy(acc_sh.at[pl.ds(r0, ROWS_PER_TILE)],
                    acc_out.at[cid].at[pl.ds(r0, ROWS_PER_TILE)])


RB = 1000


def _tc_body(relu, p0_ref, p1_ref, degt_ref, x_ref, wl_ref, wr_ref, b_ref,
             o_ref):
    deg = degt_ref[:, 0:1] + degt_ref[:, 1:2]
    inv = 1.0 / jnp.maximum(deg, 1.0)
    agg = (p0_ref[0] + p1_ref[0]) * inv
    y = (jnp.dot(agg, wl_ref[...], preferred_element_type=f32)
         + jnp.dot(x_ref[...], wr_ref[...], preferred_element_type=f32)
         + b_ref[...])
    if relu:
        y = jnp.maximum(y, 0.0)
    o_ref[...] = y


def _tc_layer(parts, degt, x_in, W_l, W_r, b, relu):
    return pl.pallas_call(
        functools.partial(_tc_body, relu),
        grid=(N // RB,),
        in_specs=[
            pl.BlockSpec((1, RB, D), lambda i: (0, i, 0)),
            pl.BlockSpec((1, RB, D), lambda i: (1, i, 0)),
            pl.BlockSpec((RB, 2), lambda i: (i, 0)),
            pl.BlockSpec((RB, D), lambda i: (i, 0)),
            pl.BlockSpec((D, H), lambda i: (0, 0)),
            pl.BlockSpec((D, H), lambda i: (0, 0)),
            pl.BlockSpec((1, H), lambda i: (0, 0)),
        ],
        out_specs=pl.BlockSpec((RB, H), lambda i: (i, 0)),
        out_shape=jax.ShapeDtypeStruct((N, H), f32),
    )(parts, parts, degt, x_in, W_l, W_r, b.reshape(1, H))


def kernel(x, edge_index, W_l0, W_r0, b0, W_l1, W_r1, b1):
    src = edge_index[0]
    dst = edge_index[1]
    pad = EPAD - E
    src_p = jnp.concatenate([src, jnp.arange(pad, dtype=i32) % N])
    dst_p = jnp.concatenate([dst, N + jnp.arange(pad, dtype=i32) % (NPAD - N)])
    src3 = src_p.reshape(NW, KC, CB)
    dst3 = dst_p.reshape(NW, KC, CB)
    z2 = jnp.zeros((NPAD, D), f32)
    z1 = jnp.zeros((NPAD,), f32)

    acc0, deg = _sc_segsum_deg(x, src3, dst3, z2, z1)
    degt = deg.T
    h = _tc_layer(acc0, degt, x, W_l0, W_r0, b0, relu=True)
    (acc1,) = _sc_segsum(h, src3, dst3, z2)
    return _tc_layer(acc1, degt, h, W_l1, W_r1, b1, relu=False)


# speedup vs baseline: 2.1143x; 1.0109x over previous
import functools

import jax
import jax.numpy as jnp
from jax import lax
from jax.experimental import pallas as pl
from jax.experimental.pallas import tpu as pltpu
from jax.experimental.pallas import tpu_sc as plsc

N = 10000
E = 320000
D = 128
H = 128

NPAD = 10240
CH = 128
NC = 2
NS = 16
NW = NC * NS
K = 80
EPAD = NW * K * CH
MC = 1
CB = MC * CH
KC = K // MC
G = 16
NG = KC // G
ROWS_PER_TILE = NPAD // NS

f32 = jnp.float32
i32 = jnp.int32

_MESH = plsc.VectorSubcoreMesh(core_axis_name="c", subcore_axis_name="s")


def _common_prologue(src_hbm, zeros2d_hbm, src_v, acc_sh, cid, sid):
    wid = cid * NS + sid
    r0 = sid * ROWS_PER_TILE
    pltpu.sync_copy(zeros2d_hbm.at[pl.ds(r0, ROWS_PER_TILE)],
                    acc_sh.at[pl.ds(r0, ROWS_PER_TILE)])
    pltpu.sync_copy(src_hbm.at[wid], src_v)


def _gather_scatter_loop(feat_hbm, dst_hbm, wid, src_v, dst_g, rows_v,
                         acc_sh, gsem, chunk_extra):
    pltpu.sync_copy(dst_hbm.at[wid], dst_g)

    def body(j, carry):
        pltpu.async_copy(feat_hbm.at[src_v.at[j]], rows_v, gsem).wait()
        if chunk_extra is not None:
            chunk_extra(j)
        pltpu.sync_copy(rows_v, acc_sh.at[dst_g.at[j]], add=True)
        return carry

    lax.fori_loop(0, KC, body, 0)


@functools.partial(
    pl.kernel,
    out_type=[jax.ShapeDtypeStruct((NC, NPAD, D), f32),
              jax.ShapeDtypeStruct((NC, NPAD), f32)],
    mesh=_MESH,
    scratch_types=[
        pltpu.VMEM((KC, CB), i32),
        pltpu.VMEM((KC, CB), i32),
        pltpu.VMEM((CB, D), f32),
        pltpu.VMEM((CB,), f32),
        pltpu.VMEM_SHARED((NPAD, D), f32),
        pltpu.VMEM_SHARED((NPAD,), f32),
        pltpu.SemaphoreType.DMA,
    ],
    name="sc_segsum_deg",
)
def _sc_segsum_deg(feat_hbm, src_hbm, dst_hbm, zeros2d_hbm, zeros1d_hbm,
                   acc_out, deg_out,
                   src_v, dst_g, rows_v, ones_v, acc_sh, deg_sh, gsem):
    cid = lax.axis_index("c")
    sid = lax.axis_index("s")
    wid = cid * NS + sid
    r0 = sid * ROWS_PER_TILE
    _common_prologue(src_hbm, zeros2d_hbm, src_v, acc_sh, cid, sid)
    pltpu.sync_copy(zeros1d_hbm.at[pl.ds(r0, ROWS_PER_TILE)],
                    deg_sh.at[pl.ds(r0, ROWS_PER_TILE)])
    for i in range(CB // 16):
        ones_v[pl.ds(i * 16, 16)] = jnp.ones((16,), f32)
    plsc.subcore_barrier()

    def count_deg(u):
        pltpu.sync_copy(ones_v, deg_sh.at[dst_g.at[u]], add=True)

    _gather_scatter_loop(feat_hbm, dst_hbm, wid, src_v, dst_g, rows_v,
                         acc_sh, gsem, count_deg)
    plsc.subcore_barrier()
    pltpu.sync_copy(acc_sh.at[pl.ds(r0, ROWS_PER_TILE)],
                    acc_out.at[cid].at[pl.ds(r0, ROWS_PER_TILE)])
    pltpu.sync_copy(deg_sh.at[pl.ds(r0, ROWS_PER_TILE)],
                    deg_out.at[cid].at[pl.ds(r0, ROWS_PER_TILE)])


@functools.partial(
    pl.kernel,
    out_type=[jax.ShapeDtypeStruct((NC, NPAD, D), f32)],
    mesh=_MESH,
    scratch_types=[
        pltpu.VMEM((KC, CB), i32),
        pltpu.VMEM((KC, CB), i32),
        pltpu.VMEM((CB, D), f32),
        pltpu.VMEM_SHARED((NPAD, D), f32),
        pltpu.SemaphoreType.DMA,
    ],
    name="sc_segsum",
)
def _sc_segsum(feat_hbm, src_hbm, dst_hbm, zeros2d_hbm,
               acc_out,
               src_v, dst_g, rows_v, acc_sh, gsem):
    cid = lax.axis_index("c")
    sid = lax.axis_index("s")
    wid = cid * NS + sid
    r0 = sid * ROWS_PER_TILE
    _common_prologue(src_hbm, zeros2d_hbm, src_v, acc_sh, cid, sid)
    plsc.subcore_barrier()
    _gather_scatter_loop(feat_hbm, dst_hbm, wid, src_v, dst_g, rows_v,
                         acc_sh, gsem, None)
    plsc.subcore_barrier()
    pltpu.sync_copy(acc_sh.at[pl.ds(r0, ROWS_PER_TILE)],
                    acc_out.at[cid].at[pl.ds(r0, ROWS_PER_TILE)])


RB = 1000


def _tc_body(relu, p0_ref, p1_ref, degt_ref, x_ref, wl_ref, wr_ref, b_ref,
             o_ref):
    deg = degt_ref[:, 0:1] + degt_ref[:, 1:2]
    inv = 1.0 / jnp.maximum(deg, 1.0)
    agg = (p0_ref[0] + p1_ref[0]) * inv
    y = (jnp.dot(agg, wl_ref[...], preferred_element_type=f32)
         + jnp.dot(x_ref[...], wr_ref[...], preferred_element_type=f32)
         + b_ref[...])
    if relu:
        y = jnp.maximum(y, 0.0)
    o_ref[...] = y


def _tc_layer(parts, degt, x_in, W_l, W_r, b, relu):
    return pl.pallas_call(
        functools.partial(_tc_body, relu),
        grid=(N // RB,),
        in_specs=[
            pl.BlockSpec((1, RB, D), lambda i: (0, i, 0)),
            pl.BlockSpec((1, RB, D), lambda i: (1, i, 0)),
            pl.BlockSpec((RB, 2), lambda i: (i, 0)),
            pl.BlockSpec((RB, D), lambda i: (i, 0)),
            pl.BlockSpec((D, H), lambda i: (0, 0)),
            pl.BlockSpec((D, H), lambda i: (0, 0)),
            pl.BlockSpec((1, H), lambda i: (0, 0)),
        ],
        out_specs=pl.BlockSpec((RB, H), lambda i: (i, 0)),
        out_shape=jax.ShapeDtypeStruct((N, H), f32),
    )(parts, parts, degt, x_in, W_l, W_r, b.reshape(1, H))


def kernel(x, edge_index, W_l0, W_r0, b0, W_l1, W_r1, b1):
    src = edge_index[0]
    dst = edge_index[1]
    pad = EPAD - E
    src_p = jnp.concatenate([src, jnp.arange(pad, dtype=i32) % N])
    dst_p = jnp.concatenate([dst, N + jnp.arange(pad, dtype=i32) % (NPAD - N)])
    src3 = src_p.reshape(NW, KC, CB)
    dst3 = dst_p.reshape(NW, KC, CB)
    z2 = jnp.zeros((NPAD, D), f32)
    z1 = jnp.zeros((NPAD,), f32)

    acc0, deg = _sc_segsum_deg(x, src3, dst3, z2, z1)
    degt = deg.T
    h = _tc_layer(acc0, degt, x, W_l0, W_r0, b0, relu=True)
    (acc1,) = _sc_segsum(h, src3, dst3, z2)
    return _tc_layer(acc1, degt, h, W_l1, W_r1, b1, relu=False)


# speedup vs baseline: 2.1172x; 1.0014x over previous
import functools

import jax
import jax.numpy as jnp
from jax import lax
from jax.experimental import pallas as pl
from jax.experimental.pallas import tpu as pltpu
from jax.experimental.pallas import tpu_sc as plsc

N = 10000
E = 320000
D = 128
H = 128

NPAD = 10240
CH = 128
NC = 2
NS = 16
NW = NC * NS
K = 80
EPAD = NW * K * CH
CB = CH
KC = K
ROWS_PER_TILE = NPAD // NS

f32 = jnp.float32
i32 = jnp.int32

_MESH = plsc.VectorSubcoreMesh(core_axis_name="c", subcore_axis_name="s")


def _common_prologue(src_hbm, zeros2d_hbm, src_v, acc_sh, cid, sid):
    wid = cid * NS + sid
    r0 = sid * ROWS_PER_TILE
    pltpu.sync_copy(zeros2d_hbm.at[pl.ds(r0, ROWS_PER_TILE)],
                    acc_sh.at[pl.ds(r0, ROWS_PER_TILE)])
    pltpu.sync_copy(src_hbm.at[wid], src_v)


def _gather_scatter_loop(feat_hbm, dst_hbm, wid, src_v, dst_g, rows_v,
                         acc_sh, gsem, chunk_extra):
    pltpu.sync_copy(dst_hbm.at[wid], dst_g)

    def body(j, carry):
        pltpu.async_copy(feat_hbm.at[src_v.at[j]], rows_v, gsem).wait()
        if chunk_extra is not None:
            chunk_extra(j)
        pltpu.sync_copy(rows_v, acc_sh.at[dst_g.at[j]], add=True)
        return carry

    lax.fori_loop(0, KC, body, 0)


@functools.partial(
    pl.kernel,
    out_type=[jax.ShapeDtypeStruct((NC, NPAD, D), f32),
              jax.ShapeDtypeStruct((NC, NPAD), f32)],
    mesh=_MESH,
    scratch_types=[
        pltpu.VMEM((KC, CB), i32),
        pltpu.VMEM((KC, CB), i32),
        pltpu.VMEM((CB, D), f32),
        pltpu.VMEM((CB,), f32),
        pltpu.VMEM_SHARED((NPAD, D), f32),
        pltpu.VMEM_SHARED((NPAD,), f32),
        pltpu.SemaphoreType.DMA,
    ],
    name="sc_segsum_deg",
)
def _sc_segsum_deg(feat_hbm, src_hbm, dst_hbm, zeros2d_hbm, zeros1d_hbm,
                   acc_out, deg_out,
                   src_v, dst_g, rows_v, ones_v, acc_sh, deg_sh, gsem):
    cid = lax.axis_index("c")
    sid = lax.axis_index("s")
    wid = cid * NS + sid
    r0 = sid * ROWS_PER_TILE
    _common_prologue(src_hbm, zeros2d_hbm, src_v, acc_sh, cid, sid)
    pltpu.sync_copy(zeros1d_hbm.at[pl.ds(r0, ROWS_PER_TILE)],
                    deg_sh.at[pl.ds(r0, ROWS_PER_TILE)])
    for i in range(CB // 16):
        ones_v[pl.ds(i * 16, 16)] = jnp.ones((16,), f32)
    plsc.subcore_barrier()

    def count_deg(u):
        pltpu.sync_copy(ones_v, deg_sh.at[dst_g.at[u]], add=True)

    _gather_scatter_loop(feat_hbm, dst_hbm, wid, src_v, dst_g, rows_v,
                         acc_sh, gsem, count_deg)
    plsc.subcore_barrier()
    pltpu.sync_copy(acc_sh.at[pl.ds(r0, ROWS_PER_TILE)],
                    acc_out.at[cid].at[pl.ds(r0, ROWS_PER_TILE)])
    pltpu.sync_copy(deg_sh.at[pl.ds(r0, ROWS_PER_TILE)],
                    deg_out.at[cid].at[pl.ds(r0, ROWS_PER_TILE)])


@functools.partial(
    pl.kernel,
    out_type=[jax.ShapeDtypeStruct((NC, NPAD, D), f32)],
    mesh=_MESH,
    scratch_types=[
        pltpu.VMEM((KC, CB), i32),
        pltpu.VMEM((KC, CB), i32),
        pltpu.VMEM((CB, D), f32),
        pltpu.VMEM_SHARED((NPAD, D), f32),
        pltpu.SemaphoreType.DMA,
    ],
    name="sc_segsum",
)
def _sc_segsum(feat_hbm, src_hbm, dst_hbm, zeros2d_hbm,
               acc_out,
               src_v, dst_g, rows_v, acc_sh, gsem):
    cid = lax.axis_index("c")
    sid = lax.axis_index("s")
    wid = cid * NS + sid
    r0 = sid * ROWS_PER_TILE
    _common_prologue(src_hbm, zeros2d_hbm, src_v, acc_sh, cid, sid)
    plsc.subcore_barrier()
    _gather_scatter_loop(feat_hbm, dst_hbm, wid, src_v, dst_g, rows_v,
                         acc_sh, gsem, None)
    plsc.subcore_barrier()
    pltpu.sync_copy(acc_sh.at[pl.ds(r0, ROWS_PER_TILE)],
                    acc_out.at[cid].at[pl.ds(r0, ROWS_PER_TILE)])


RB = 1000


def _tc_body(relu, p0_ref, p1_ref, degt_ref, x_ref, wl_ref, wr_ref, b_ref,
             o_ref):
    deg = degt_ref[:, 0:1] + degt_ref[:, 1:2]
    inv = 1.0 / jnp.maximum(deg, 1.0)
    agg = (p0_ref[0] + p1_ref[0]) * inv
    y = (jnp.dot(agg, wl_ref[...], preferred_element_type=f32)
         + jnp.dot(x_ref[...], wr_ref[...], preferred_element_type=f32)
         + b_ref[...])
    if relu:
        y = jnp.maximum(y, 0.0)
    o_ref[...] = y


def _tc_layer(parts, degt, x_in, W_l, W_r, b, relu):
    return pl.pallas_call(
        functools.partial(_tc_body, relu),
        grid=(N // RB,),
        in_specs=[
            pl.BlockSpec((1, RB, D), lambda i: (0, i, 0)),
            pl.BlockSpec((1, RB, D), lambda i: (1, i, 0)),
            pl.BlockSpec((RB, 2), lambda i: (i, 0)),
            pl.BlockSpec((RB, D), lambda i: (i, 0)),
            pl.BlockSpec((D, H), lambda i: (0, 0)),
            pl.BlockSpec((D, H), lambda i: (0, 0)),
            pl.BlockSpec((1, H), lambda i: (0, 0)),
        ],
        out_specs=pl.BlockSpec((RB, H), lambda i: (i, 0)),
        out_shape=jax.ShapeDtypeStruct((N, H), f32),
    )(parts, parts, degt, x_in, W_l, W_r, b.reshape(1, H))


def kernel(x, edge_index, W_l0, W_r0, b0, W_l1, W_r1, b1):
    src = edge_index[0]
    dst = edge_index[1]
    pad = EPAD - E
    src_p = jnp.concatenate([src, jnp.arange(pad, dtype=i32) % N])
    dst_p = jnp.concatenate([dst, N + jnp.arange(pad, dtype=i32) % (NPAD - N)])
    src3 = src_p.reshape(NW, KC, CB)
    dst3 = dst_p.reshape(NW, KC, CB)
    z2 = jnp.zeros((NPAD, D), f32)
    z1 = jnp.zeros((NPAD,), f32)

    acc0, deg = _sc_segsum_deg(x, src3, dst3, z2, z1)
    degt = deg.T
    h = _tc_layer(acc0, degt, x, W_l0, W_r0, b0, relu=True)
    (acc1,) = _sc_segsum(h, src3, dst3, z2)
    return _tc_layer(acc1, degt, h, W_l1, W_r1, b1, relu=False)
